# baseline (speedup 1.0000x reference)
import functools

import jax
import jax.numpy as jnp
from jax import lax
from jax.experimental import pallas as pl
from jax.experimental.pallas import tpu as pltpu
from jax.experimental.pallas import tpu_sc as plsc

BATCH = 1024
SEQ_LEN = 2048
ALPHABET = 4
N = BATCH * SEQ_LEN

NUM_CORES = 2
NUM_SUBCORES = 16
NW = NUM_CORES * NUM_SUBCORES
NSLAB = BATCH // 8
SLABS_PER_W = NSLAB // NW
IN_SLAB = 8 * SEQ_LEN
OUT_Q = 2 * SEQ_LEN * 4
NOB = 4
LANES = 16
NT = SEQ_LEN // 128


def _body(in_hbm, out_hbm, in_v, out_v, in_sem, out_sem):
    wid = lax.axis_index("s") * NUM_CORES + lax.axis_index("c")
    slab0 = wid * SLABS_PER_W

    one = jnp.full((LANES,), 1.0, jnp.float32)
    zero = jnp.zeros((LANES,), jnp.float32)

    def start_in(i):
        return pltpu.async_copy(
            in_hbm.at[pl.ds((slab0 + i) * IN_SLAB, IN_SLAB)],
            in_v.at[i % 2], in_sem)

    def start_out(i, q, ob):
        return pltpu.async_copy(
            out_v.at[ob],
            out_hbm.at[pl.ds(((slab0 + i) * 4 + q) * OUT_Q, OUT_Q)],
            out_sem)

    def compute_quarter(ib, q, ob):
        in_ref = in_v.at[ib]
        out_ref = out_v.at[ob]

        def jbody(j, carry):
            t = j // 2
            bp = j % 2
            bin_ = (t * 8 + 2 * q + bp) * 128
            bout = bp * 8192 + t * 512
            for k in range(8):
                vals = in_ref[pl.ds(bin_ + k * LANES, LANES)]
                for c in range(ALPHABET):
                    out_ref[pl.ds(bout + c * 128 + k * LANES, LANES)] = (
                        jnp.where(vals == c, one, zero))
            return carry

        lax.fori_loop(0, 2 * NT, jbody, 0, unroll=2)

    in_copies = [start_in(0)]
    out_copies = []
    step = 0
    for i in range(SLABS_PER_W):
        if i + 1 < SLABS_PER_W:
            in_copies.append(start_in(i + 1))
        in_copies[i].wait()
        for q in range(4):
            ob = step % NOB
            if step >= NOB:
                out_copies[step - NOB].wait()
            compute_quarter(i % 2, q, ob)
            out_copies.append(start_out(i, q, ob))
            step += 1
    for s in range(step - NOB, step):
        out_copies[s].wait()


@jax.jit
def _one_hot_sc(flat_in):
    mesh = plsc.VectorSubcoreMesh(
        core_axis_name="c", subcore_axis_name="s",
        num_cores=NUM_CORES, num_subcores=NUM_SUBCORES)
    return pl.kernel(
        _body,
        out_type=jax.ShapeDtypeStruct((N * 4,), jnp.float32),
        mesh=mesh,
        scratch_types=[
            pltpu.VMEM((2, IN_SLAB), jnp.int32),
            pltpu.VMEM((NOB, OUT_Q), jnp.float32),
            pltpu.SemaphoreType.DMA,
            pltpu.SemaphoreType.DMA,
        ],
    )(flat_in)


def kernel(input):
    flat_in = (input.reshape(NSLAB, 8, NT, 128)
               .transpose(0, 2, 1, 3)
               .reshape(N))
    out_flat = _one_hot_sc(flat_in)
    return (out_flat.reshape(BATCH, NT, ALPHABET, 128)
            .transpose(0, 1, 3, 2)
            .reshape(BATCH, SEQ_LEN, ALPHABET))


# speedup vs baseline: 1.1691x; 1.1691x over previous
import functools

import jax
import jax.numpy as jnp
from jax import lax
from jax.experimental import pallas as pl
from jax.experimental.pallas import tpu as pltpu
from jax.experimental.pallas import tpu_sc as plsc

BATCH = 1024
SEQ_LEN = 2048
ALPHABET = 4
N = BATCH * SEQ_LEN

NUM_CORES = 2
NUM_SUBCORES = 16
NW = NUM_CORES * NUM_SUBCORES
NSLAB = BATCH // 8
SLABS_PER_W = NSLAB // NW
IN_SLAB = 8 * SEQ_LEN
OUT_Q = 2 * SEQ_LEN * 4
NOB = 4
LANES = 16
NT = SEQ_LEN // 128


def _body(in_hbm, out_hbm, in_v, out_v, in_sem, out_sem):
    wid = lax.axis_index("s") * NUM_CORES + lax.axis_index("c")
    slab0 = wid * SLABS_PER_W

    one = jnp.full((LANES,), 1.0, jnp.float32)
    zero = jnp.zeros((LANES,), jnp.float32)

    def start_in(i):
        return pltpu.async_copy(
            in_hbm.at[pl.ds((slab0 + i) * IN_SLAB, IN_SLAB)],
            in_v.at[i % 2], in_sem)

    def start_out(i, q, ob):
        return pltpu.async_copy(
            out_v.at[ob],
            out_hbm.at[pl.ds(((slab0 + i) * 4 + q) * OUT_Q, OUT_Q)],
            out_sem)

    def compute_quarter(ib, q, ob):
        in_ref = in_v.at[ib]
        out_ref = out_v.at[ob]

        def jbody(j, carry):
            t = j // 2
            bp = j % 2
            bin_ = (t * 8 + 2 * q + bp) * 128
            bout = bp * 8192 + t * 512
            for k in range(8):
                vals = in_ref[pl.ds(bin_ + k * LANES, LANES)]
                for c in range(ALPHABET):
                    out_ref[pl.ds(bout + c * 128 + k * LANES, LANES)] = (
                        jnp.where(vals == c, one, zero))
            return carry

        lax.fori_loop(0, 2 * NT, jbody, 0)

    in_copies = [start_in(0)]
    out_copies = []
    step = 0
    for i in range(SLABS_PER_W):
        if i + 1 < SLABS_PER_W:
            in_copies.append(start_in(i + 1))
        in_copies[i].wait()
        for q in range(4):
            ob = step % NOB
            if step >= NOB:
                out_copies[step - NOB].wait()
            compute_quarter(i % 2, q, ob)
            out_copies.append(start_out(i, q, ob))
            step += 1
    for s in range(step - NOB, step):
        out_copies[s].wait()


@jax.jit
def _one_hot_sc(flat_in):
    mesh = plsc.VectorSubcoreMesh(
        core_axis_name="c", subcore_axis_name="s",
        num_cores=NUM_CORES, num_subcores=NUM_SUBCORES)
    return pl.kernel(
        _body,
        out_type=jax.ShapeDtypeStruct((N * 4,), jnp.float32),
        mesh=mesh,
        scratch_types=[
            pltpu.VMEM((2, IN_SLAB), jnp.int32),
            pltpu.VMEM((NOB, OUT_Q), jnp.float32),
            pltpu.SemaphoreType.DMA,
            pltpu.SemaphoreType.DMA,
        ],
    )(flat_in)


def kernel(input):
    flat_in = (input.reshape(NSLAB, 8, NT, 128)
               .transpose(0, 2, 1, 3)
               .reshape(N))
    out_flat = _one_hot_sc(flat_in)
    return (out_flat.reshape(BATCH, NT, ALPHABET, 128)
            .transpose(0, 1, 3, 2)
            .reshape(BATCH, SEQ_LEN, ALPHABET))


# speedup vs baseline: 1.3597x; 1.1630x over previous
import functools

import jax
import jax.numpy as jnp
from jax import lax
from jax.experimental import pallas as pl
from jax.experimental.pallas import tpu as pltpu
from jax.experimental.pallas import tpu_sc as plsc

BATCH = 1024
SEQ_LEN = 2048
ALPHABET = 4
N = BATCH * SEQ_LEN

NUM_CORES = 2
NUM_SUBCORES = 16
NW = NUM_CORES * NUM_SUBCORES
NSLAB = BATCH // 8
SLABS_PER_W = NSLAB // NW
IN_SLAB = 8 * SEQ_LEN
OUT_Q = 2 * SEQ_LEN * 4
NOB = 4
LANES = 16
NT = SEQ_LEN // 128


def _body(in_hbm, out_hbm, in_v, out_v, in_sem, out_sem):
    wid = lax.axis_index("s") * NUM_CORES + lax.axis_index("c")
    slab0 = wid * SLABS_PER_W

    one = jnp.full((LANES,), 1.0, jnp.float32)
    zero = jnp.zeros((LANES,), jnp.float32)

    def start_in(i):
        return pltpu.async_copy(
            in_hbm.at[pl.ds((slab0 + i) * IN_SLAB, IN_SLAB)],
            in_v.at[i % 2], in_sem)

    OUT_CHUNK = OUT_Q // (2 * NT)

    def drain_out_quarter():
        pltpu.make_async_copy(
            out_v.at[0], out_hbm.at[pl.ds(0, OUT_Q)], out_sem).wait()

    def compute_quarter(ib, q, ob, prev):
        in_ref = in_v.at[ib]
        out_ref = out_v.at[ob]

        def jbody(j, carry):
            if prev is not None:
                pob, poff = prev
                pltpu.async_copy(
                    out_v.at[pob, pl.ds(j * OUT_CHUNK, OUT_CHUNK)],
                    out_hbm.at[pl.ds(poff + j * OUT_CHUNK, OUT_CHUNK)],
                    out_sem)
            t = j // 2
            bp = j % 2
            bin_ = (t * 8 + 2 * q + bp) * 128
            bout = bp * 8192 + t * 512
            for k in range(8):
                vals = in_ref[pl.ds(bin_ + k * LANES, LANES)]
                for c in range(ALPHABET):
                    out_ref[pl.ds(bout + c * 128 + k * LANES, LANES)] = (
                        jnp.where(vals == c, one, zero))
            return carry

        lax.fori_loop(0, 2 * NT, jbody, 0)

    in_copies = [start_in(0)]
    step = 0
    prev = None
    for i in range(SLABS_PER_W):
        if i + 1 < SLABS_PER_W:
            in_copies.append(start_in(i + 1))
        in_copies[i].wait()
        for q in range(4):
            ob = step % NOB
            if step >= NOB:
                drain_out_quarter()
            compute_quarter(i % 2, q, ob, prev)
            prev = (ob, ((slab0 + i) * 4 + q) * OUT_Q)
            step += 1
    pob, poff = prev
    pltpu.async_copy(out_v.at[pob], out_hbm.at[pl.ds(poff, OUT_Q)], out_sem)
    for _ in range(NOB):
        drain_out_quarter()


@jax.jit
def _one_hot_sc(flat_in):
    mesh = plsc.VectorSubcoreMesh(
        core_axis_name="c", subcore_axis_name="s",
        num_cores=NUM_CORES, num_subcores=NUM_SUBCORES)
    return pl.kernel(
        _body,
        out_type=jax.ShapeDtypeStruct((N * 4,), jnp.float32),
        mesh=mesh,
        scratch_types=[
            pltpu.VMEM((2, IN_SLAB), jnp.int32),
            pltpu.VMEM((NOB, OUT_Q), jnp.float32),
            pltpu.SemaphoreType.DMA,
            pltpu.SemaphoreType.DMA,
        ],
    )(flat_in)


def kernel(input):
    flat_in = (input.reshape(NSLAB, 8, NT, 128)
               .transpose(0, 2, 1, 3)
               .reshape(N))
    out_flat = _one_hot_sc(flat_in)
    return (out_flat.reshape(BATCH, NT, ALPHABET, 128)
            .transpose(0, 1, 3, 2)
            .reshape(BATCH, SEQ_LEN, ALPHABET))


# speedup vs baseline: 1.9446x; 1.4302x over previous
import functools

import jax
import jax.numpy as jnp
from jax import lax
from jax.experimental import pallas as pl
from jax.experimental.pallas import tpu as pltpu
from jax.experimental.pallas import tpu_sc as plsc

BATCH = 1024
SEQ_LEN = 2048
ALPHABET = 4
N = BATCH * SEQ_LEN

NUM_CORES = 2
NUM_SUBCORES = 16
NW = NUM_CORES * NUM_SUBCORES
NSLAB = BATCH // 8
SLABS_PER_W = NSLAB // NW
IN_SLAB = 8 * SEQ_LEN
OUT_Q = 2 * SEQ_LEN * 4
NOB = 4
LANES = 16
NT = SEQ_LEN // 128


def _body(in_hbm, out_hbm, in_v, out_v, in_sem, out_sem):
    wid = lax.axis_index("s") * NUM_CORES + lax.axis_index("c")
    slab0 = wid * SLABS_PER_W

    one = jnp.full((LANES,), 1.0, jnp.float32)
    zero = jnp.zeros((LANES,), jnp.float32)

    OUT_CHUNK = OUT_Q // NT
    IN_CHUNK = IN_SLAB // NT

    def drain_out_quarter():
        pltpu.make_async_copy(
            out_v.at[0], out_hbm.at[pl.ds(0, OUT_Q)], out_sem).wait()

    def drain_in_slab():
        pltpu.make_async_copy(
            in_hbm.at[pl.ds(0, IN_SLAB)], in_v.at[0], in_sem).wait()

    def compute_quarter(ib, q, ob, prev, pref):
        in_ref = in_v.at[ib]
        out_ref = out_v.at[ob]

        def tbody(t, carry):
            if prev is not None:
                pob, poff = prev
                pltpu.async_copy(
                    out_v.at[pob, pl.ds(t * OUT_CHUNK, OUT_CHUNK)],
                    out_hbm.at[pl.ds(poff + t * OUT_CHUNK, OUT_CHUNK)],
                    out_sem)
            if pref is not None:
                pltpu.async_copy(
                    in_hbm.at[pl.ds(pref * IN_SLAB + t * IN_CHUNK, IN_CHUNK)],
                    in_v.at[1 - ib, pl.ds(t * IN_CHUNK, IN_CHUNK)],
                    in_sem)
            bin_, bout = carry
            bin_ = pl.multiple_of(bin_, 128)
            bout = pl.multiple_of(bout, 512)
            for bp in range(2):
                for k in range(8):
                    vals = in_ref[pl.ds(bin_ + bp * 128 + k * LANES, LANES)]
                    for c in range(ALPHABET):
                        out_ref[pl.ds(bout + bp * 8192 + c * 128 + k * LANES,
                                      LANES)] = jnp.where(vals == c, one, zero)
            return (bin_ + 1024, bout + 512)

        lax.fori_loop(0, NT, tbody, (2 * q * 128, 0))

    pltpu.async_copy(
        in_hbm.at[pl.ds(slab0 * IN_SLAB, IN_SLAB)], in_v.at[0], in_sem)
    step = 0
    prev = None
    for i in range(SLABS_PER_W):
        drain_in_slab()
        for q in range(4):
            ob = step % NOB
            if step >= NOB:
                drain_out_quarter()
            pref = (slab0 + i + 1) if (q == 0 and i + 1 < SLABS_PER_W) else None
            compute_quarter(i % 2, q, ob, prev, pref)
            prev = (ob, ((slab0 + i) * 4 + q) * OUT_Q)
            step += 1
    pob, poff = prev
    pltpu.async_copy(out_v.at[pob], out_hbm.at[pl.ds(poff, OUT_Q)], out_sem)
    for _ in range(NOB):
        drain_out_quarter()


@jax.jit
def _one_hot_sc(flat_in):
    mesh = plsc.VectorSubcoreMesh(
        core_axis_name="c", subcore_axis_name="s",
        num_cores=NUM_CORES, num_subcores=NUM_SUBCORES)
    return pl.kernel(
        _body,
        out_type=jax.ShapeDtypeStruct((N * 4,), jnp.float32),
        mesh=mesh,
        scratch_types=[
            pltpu.VMEM((2, IN_SLAB), jnp.int32),
            pltpu.VMEM((NOB, OUT_Q), jnp.float32),
            pltpu.SemaphoreType.DMA,
            pltpu.SemaphoreType.DMA,
        ],
    )(flat_in)


def kernel(input):
    flat_in = (input.reshape(NSLAB, 8, NT, 128)
               .transpose(0, 2, 1, 3)
               .reshape(N))
    out_flat = _one_hot_sc(flat_in)
    return (out_flat.reshape(BATCH, NT, ALPHABET, 128)
            .transpose(0, 1, 3, 2)
            .reshape(BATCH, SEQ_LEN, ALPHABET))
